# baseline (speedup 1.0000x reference)
import functools

import jax
import jax.numpy as jnp
from jax import lax
from jax.experimental import pallas as pl
from jax.experimental.pallas import tpu as pltpu
from jax.experimental.pallas import tpu_sc as plsc

B = 16384
D = 128
C = 100000
ALPHA = 0.1

NC = 2
NS = 16
NW = NC * NS
HALF = B // NC
PT = B // NW
CH = 128
NCH = PT // CH
CW = 64
NCW = PT // CW
NV = D // 16

_mesh = plsc.VectorSubcoreMesh(
    core_axis_name="c", subcore_axis_name="s", num_cores=NC, num_subcores=NS)
_params = pltpu.CompilerParams(
    needs_layout_passes=False, use_tc_tiling_on_sc=False)


def _zero_rows(buf, rows, width):
    def body(r):
        for v in range(width // 16):
            buf[r, pl.ds(v * 16, 16)] = jnp.zeros((16,), buf.dtype)
    pl.loop(0, rows)(body)


@functools.partial(
    pl.kernel,
    out_type=(
        jax.ShapeDtypeStruct((NC * C, 16), jnp.int32),
        jax.ShapeDtypeStruct((B, D), jnp.float32),
        jax.ShapeDtypeStruct((B, 16), jnp.float32),
        jax.ShapeDtypeStruct((B // CH, CH), jnp.int32),
    ),
    mesh=_mesh,
    compiler_params=_params,
    scratch_types=(
        pltpu.VMEM_SHARED((HALF, D), jnp.float32),
        pltpu.VMEM_SHARED((HALF, 16), jnp.float32),
        pltpu.VMEM((CH, D), jnp.float32),
        pltpu.VMEM((CH, 16), jnp.float32),
        pltpu.VMEM((NCH, CH), jnp.int32),
        pltpu.VMEM((NCH, CH), jnp.int32),
        pltpu.VMEM((NCH, CH), jnp.int32),
        pltpu.VMEM((NCH, CH), jnp.int32),
        pltpu.VMEM((CH, 16), jnp.int32),
        pltpu.VMEM((CH, 16), jnp.int32),
    ),
)
def _l1(features, labels2, rep_out, sums_out, meta_out, slotg_out,
        sums_sp, meta_sp, feat_v, meta_v, lab_v, laboff_v, slotl_v, slotg_v,
        irows, srows):
    c = lax.axis_index("c")
    s = lax.axis_index("s")
    half_base = c * HALF
    my = half_base + s * PT
    row0 = (c * NS + s) * NCH

    _zero_rows(feat_v, CH, D)
    _zero_rows(meta_v, CH, 16)
    for k in range(NCH):
        pltpu.sync_copy(feat_v, sums_sp.at[pl.ds(s * PT + k * CH, CH)])
        pltpu.sync_copy(meta_v, meta_sp.at[pl.ds(s * PT + k * CH, CH)])

    iota16 = lax.iota(jnp.int32, 16)
    zeros16 = jnp.zeros((16,), jnp.int32)

    for j in range(CH // 16):
        plsc.store_scatter(meta_v, [iota16 + j * 16, zeros16],
                           jnp.ones((16,), jnp.float32))

    pltpu.sync_copy(labels2.at[pl.ds(row0, NCH)], lab_v)
    for k in range(NCH):
        for v in range(CH // 16):
            sl = pl.ds(v * 16, 16)
            laboff_v[k, sl] = lab_v[k, sl] + c * C

    for k in range(NCH):
        def put_idx(r, _k=k):
            irows[r, pl.ds(0, 16)] = (
                jnp.zeros((16,), jnp.int32) + (my + _k * CH + r))
        pl.loop(0, CH)(put_idx)
        pltpu.sync_copy(irows, rep_out.at[laboff_v.at[k]])

    plsc.subcore_barrier()

    for k in range(NCH):
        pltpu.sync_copy(rep_out.at[laboff_v.at[k]], srows)
        for j in range(CH // 16):
            sl = pl.ds(j * 16, 16)
            col0 = plsc.load_gather(srows, [iota16 + j * 16, zeros16])
            slotg_v[k, sl] = col0
            slotl_v[k, sl] = col0 - half_base
    pltpu.sync_copy(slotg_v, slotg_out.at[pl.ds(row0, NCH)])

    for k in range(NCH):
        pltpu.sync_copy(features.at[pl.ds(my + k * CH, CH)], feat_v)
        pltpu.sync_copy(feat_v, sums_sp.at[slotl_v.at[k]], add=True)

        for j in range(CH // 16):
            sl = pl.ds(j * 16, 16)
            rid = iota16 + j * 16
            gidx = slotg_v[k, sl]
            own = iota16 + (my + k * CH + j * 16)
            labf = lab_v[k, sl].astype(jnp.float32)
            val = jnp.where(gidx == own, labf, jnp.float32(0.0))
            plsc.store_scatter(meta_v, [rid, zeros16 + 1], val)
        pltpu.sync_copy(meta_v, meta_sp.at[slotl_v.at[k]], add=True)

    plsc.subcore_barrier()

    for k in range(NCH):
        rows = pl.ds(s * PT + k * CH, CH)
        out_rows = pl.ds(half_base + s * PT + k * CH, CH)
        pltpu.sync_copy(sums_sp.at[rows], feat_v)
        pltpu.sync_copy(feat_v, sums_out.at[out_rows])
        pltpu.sync_copy(meta_sp.at[rows], meta_v)
        pltpu.sync_copy(meta_v, meta_out.at[out_rows])


@functools.partial(
    pl.kernel,
    out_type=jax.ShapeDtypeStruct((NW, CH), jnp.float32),
    mesh=_mesh,
    compiler_params=_params,
    scratch_types=(
        pltpu.VMEM((NCW, CW), jnp.int32),
        pltpu.VMEM((NCW, CW), jnp.int32),
        pltpu.VMEM((NCW, CW), jnp.int32),
        pltpu.VMEM((2, CW), jnp.int32),
        pltpu.VMEM((2, CW, 16), jnp.int32),
        pltpu.VMEM((2, CW, D), jnp.float32),
        pltpu.VMEM((2, CW, D), jnp.float32),
        pltpu.VMEM((2, CW, 16), jnp.float32),
        pltpu.VMEM((2, CW, 16), jnp.float32),
        pltpu.VMEM((2, CW, D), jnp.float32),
        pltpu.VMEM((2, CW, D), jnp.float32),
        pltpu.VMEM((2, CW, D), jnp.float32),
        pltpu.VMEM((1, CH), jnp.float32),
        pltpu.VMEM((2, 16), jnp.float32),
        pltpu.SemaphoreType.DMA((2,)),
        pltpu.SemaphoreType.DMA((2,)),
        pltpu.SemaphoreType.DMA((2,)),
    ),
)
def _l2(features, labels3, centers, rep_in, sums_in, meta_in, slotg3, cpy,
        partials,
        lab_v, labO_v, slotg_v, idxO, srowsO, sumsN, sumsO, metaN, metaO,
        crows, frows, obuf, pacc, scbuf, semA, semB, semS):
    c = lax.axis_index("c")
    s = lax.axis_index("s")
    wid = c * NS + s
    my = wid * PT
    row0 = wid * NCW
    oth = 1 - c
    iota16 = lax.iota(jnp.int32, 16)
    zeros16 = jnp.zeros((16,), jnp.int32)

    pltpu.sync_copy(labels3.at[pl.ds(row0, NCW)], lab_v)
    pltpu.sync_copy(slotg3.at[pl.ds(row0, NCW)], slotg_v)
    for k in range(NCW):
        for v in range(CW // 16):
            sl = pl.ds(v * 16, 16)
            labO_v[k, sl] = lab_v[k, sl] + oth * C

    lo = oth * HALF
    hi = lo + (HALF - 1)

    descA = [None] * NCW
    descB = [None] * NCW
    descS = [None] * NCW

    def issue_a(k):
        b = k % 2
        descA[k] = pltpu.make_async_copy(
            rep_in.at[labO_v.at[k]], srowsO.at[b], semA.at[b])
        descA[k].start()

    def issue_b(k):
        b = k % 2
        descA[k].wait()
        for j in range(CW // 16):
            g = plsc.load_gather(srowsO.at[b], [iota16 + j * 16, zeros16])
            idxO[b, pl.ds(j * 16, 16)] = jnp.minimum(jnp.maximum(g, lo), hi)
        descB[k] = [
            pltpu.make_async_copy(sums_in.at[slotg_v.at[k]], sumsN.at[b],
                                  semB.at[b]),
            pltpu.make_async_copy(meta_in.at[slotg_v.at[k]], metaN.at[b],
                                  semB.at[b]),
            pltpu.make_async_copy(sums_in.at[idxO.at[b]], sumsO.at[b],
                                  semB.at[b]),
            pltpu.make_async_copy(meta_in.at[idxO.at[b]], metaO.at[b],
                                  semB.at[b]),
            pltpu.make_async_copy(centers.at[lab_v.at[k]], crows.at[b],
                                  semB.at[b]),
            pltpu.make_async_copy(features.at[pl.ds(my + k * CW, CW)],
                                  frows.at[b], semB.at[b]),
        ]
        for d in descB[k]:
            d.start()

    def run_c(k, acc):
        b = k % 2
        if k >= 2:
            descS[k - 2].wait()
        for d in descB[k]:
            d.wait()

        def group(j, acc2, _k=k, _b=b):
            rid = iota16 + j * 16
            labf = plsc.load_gather(
                lab_v, [zeros16 + _k, rid]).astype(jnp.float32)
            cN = plsc.load_gather(metaN, [zeros16 + _b, rid, zeros16])
            labO = plsc.load_gather(metaO, [zeros16 + _b, rid, zeros16 + 1])
            cO = plsc.load_gather(metaO, [zeros16 + _b, rid, zeros16])
            wO = jnp.where(labO == labf, jnp.float32(1.0), jnp.float32(0.0))
            cnt = cN + wO * cO
            scv = jnp.float32(ALPHA) / cnt
            scbuf[0, pl.ds(0, 16)] = scv
            scbuf[1, pl.ds(0, 16)] = scv * wO

            def row(i, acc3):
                r = j * 16 + i
                an = plsc.load_gather(scbuf, [zeros16, zeros16 + i])
                ao = plsc.load_gather(scbuf, [zeros16 + 1, zeros16 + i])
                for v in range(NV):
                    sl = pl.ds(v * 16, 16)
                    cv = crows[_b, r, sl]
                    fv = frows[_b, r, sl]
                    obuf[_b, r, sl] = (cv * jnp.float32(1.0 - ALPHA)
                                       + an * sumsN[_b, r, sl]
                                       + ao * sumsO[_b, r, sl])
                    dv = fv - cv
                    acc3 = acc3 + dv * dv
                return acc3
            return pl.loop(0, 16, init_carry=acc2)(row)
        acc = pl.loop(0, CW // 16, init_carry=acc)(group)

        descS[k] = pltpu.make_async_copy(
            obuf.at[b], cpy.at[lab_v.at[k]], semS.at[b])
        descS[k].start()
        return acc

    issue_a(0)
    issue_a(1)
    issue_b(0)
    acc = jnp.zeros((16,), jnp.float32)
    for k in range(NCW):
        if k + 2 < NCW:
            issue_a(k + 2)
        if k + 1 < NCW:
            issue_b(k + 1)
        acc = run_c(k, acc)
    descS[NCW - 2].wait()
    descS[NCW - 1].wait()

    for v in range(CH // 16):
        pacc[0, pl.ds(v * 16, 16)] = jnp.zeros((16,), jnp.float32)
    pacc[0, pl.ds(0, 16)] = acc
    pltpu.sync_copy(pacc, partials.at[pl.ds(wid, 1)])


def _l3_body(p_ref, o_ref):
    o_ref[...] = jnp.broadcast_to(jnp.sum(p_ref[...]) * (1.0 / B), (8, 128))


def kernel(features, labels, centers):
    labels = labels.astype(jnp.int32)
    labels2 = labels.reshape(B // CH, CH)
    labels3 = labels.reshape(B // CW, CW)

    cpy = jax.new_ref(centers)
    rep, sums, meta, slotg = _l1(features, labels2)
    slotg3 = slotg.reshape(B // CW, CW)

    partials = _l2(features, labels3, centers, rep, sums, meta, slotg3, cpy)
    new_centers = cpy[...]

    lossmat = pl.pallas_call(
        _l3_body,
        out_shape=jax.ShapeDtypeStruct((8, 128), jnp.float32),
    )(partials)
    return lossmat[0, 0], new_centers
